# Initial kernel scaffold; baseline (speedup 1.0000x reference)
import jax
import jax.numpy as jnp
from jax.experimental import pallas as pl


def kernel(inputs, emb_w):
    raise NotImplementedError("write your pallas kernel here")


# speedup vs baseline: 1.0155x; 1.0155x over previous
import functools

import jax
import jax.numpy as jnp
from jax import lax
from jax.experimental import pallas as pl
from jax.experimental.pallas import tpu as pltpu
from jax.experimental.pallas import tpu_sc as plsc

K_CODES = 8192
DIM = 256
N_TOK = 8192
COMMIT = 0.25

TN = 512
TK = 512
TN_E = 256
NT_E = N_TOK // TN_E
TN_F = 512
NT_F = N_TOK // TN_F


def _argmin_body(x_ref, embt_ref, xsq_ref, esq_ref, idx_ref):
    x = x_ref[...]
    xsq = xsq_ref[...]
    best = jnp.full((TN, 1), jnp.inf, dtype=jnp.float32)
    bidx = jnp.zeros((TN, 1), dtype=jnp.int32)
    for j in range(K_CODES // TK):
        e = embt_ref[:, j * TK:(j + 1) * TK]
        mm = jnp.dot(x, e, preferred_element_type=jnp.float32)
        d = (xsq + esq_ref[:, j * TK:(j + 1) * TK]) - 2.0 * mm
        lmin = jnp.min(d, axis=1, keepdims=True)
        ii = lax.broadcasted_iota(jnp.int32, (TN, TK), 1)
        larg = jnp.min(jnp.where(d == lmin, ii, K_CODES), axis=1,
                       keepdims=True) + j * TK
        upd = lmin < best
        bidx = jnp.where(upd, larg, bidx)
        best = jnp.where(upd, lmin, best)
    idx_ref[...] = bidx


def _onehot_body(idx_ref, enc_ref, pc_ref):
    bidx = idx_ref[...]
    ch = 1024
    for k in range(K_CODES // ch):
        ii = lax.broadcasted_iota(jnp.int32, (TN_E, ch), 1) + k * ch
        enc = (ii == bidx).astype(jnp.float32)
        enc_ref[:, k * ch:(k + 1) * ch] = enc
        pc_ref[0, 0, k * ch:(k + 1) * ch] = jnp.sum(enc, axis=0)


def _final_body(x_ref, q_ref, pc_ref, qst_ref, loss_ref, perp_ref, acc_ref):
    t = pl.program_id(0)
    x = x_ref[...]
    q = q_ref[...]
    diff = q - x
    qst_ref[...] = x + diff
    psum = jnp.sum(diff * diff)
    prev = jnp.where(t == 0, 0.0, acc_ref[0, 0])
    acc_ref[0, 0] = prev + psum

    @pl.when(t == NT_F - 1)
    def _():
        m = acc_ref[0, 0] * (1.0 / (N_TOK * DIM))
        loss_ref[...] = (m + COMMIT * m)[None, None]
        pc = pc_ref[...].reshape(NT_E, K_CODES)
        counts = jnp.sum(pc, axis=0)
        p = counts * (1.0 / N_TOK)
        ent = jnp.sum(p * jnp.log(p + 1e-10))
        perp_ref[...] = jnp.exp(-ent)[None, None]


_NW = 32
_B_PER_W = N_TOK // _NW


def _sc_gather(emb_w, idx):
    mesh = plsc.VectorSubcoreMesh(core_axis_name="c", subcore_axis_name="s")

    @functools.partial(
        pl.kernel, mesh=mesh,
        out_type=jax.ShapeDtypeStruct((N_TOK, DIM), jnp.float32),
        scratch_types=[
            pltpu.VMEM((_B_PER_W,), jnp.int32),
            pltpu.VMEM((_B_PER_W, DIM), jnp.float32),
            pltpu.SemaphoreType.DMA,
        ],
    )
    def k(table_hbm, idx_hbm, out_hbm, idx_v, rows_v, sem):
        wid = lax.axis_index("s") * 2 + lax.axis_index("c")
        base = wid * _B_PER_W
        pltpu.sync_copy(idx_hbm.at[pl.ds(base, _B_PER_W)], idx_v)
        pltpu.async_copy(table_hbm.at[idx_v], rows_v, sem).wait()
        pltpu.sync_copy(rows_v, out_hbm.at[pl.ds(base, _B_PER_W)])

    return k(emb_w, idx)


def kernel(inputs, emb_w):
    x = jnp.transpose(inputs, (0, 2, 3, 1))
    in_shape = x.shape
    flat = x.reshape(-1, DIM)
    xsq = jnp.sum(flat ** 2, axis=1, keepdims=True)
    esq = jnp.sum(emb_w ** 2, axis=1)[None, :]
    embt = emb_w.T

    idx2 = pl.pallas_call(
        _argmin_body,
        grid=(N_TOK // TN,),
        in_specs=[
            pl.BlockSpec((TN, DIM), lambda i: (i, 0)),
            pl.BlockSpec((DIM, K_CODES), lambda i: (0, 0)),
            pl.BlockSpec((TN, 1), lambda i: (i, 0)),
            pl.BlockSpec((1, K_CODES), lambda i: (0, 0)),
        ],
        out_specs=pl.BlockSpec((TN, 1), lambda i: (i, 0)),
        out_shape=jax.ShapeDtypeStruct((N_TOK, 1), jnp.int32),
        compiler_params=pltpu.CompilerParams(
            dimension_semantics=("parallel",)),
    )(flat, embt, xsq, esq)

    enc, pc = pl.pallas_call(
        _onehot_body,
        grid=(NT_E,),
        in_specs=[pl.BlockSpec((TN_E, 1), lambda i: (i, 0))],
        out_specs=[
            pl.BlockSpec((TN_E, K_CODES), lambda i: (i, 0)),
            pl.BlockSpec((1, 1, K_CODES), lambda i: (i, 0, 0)),
        ],
        out_shape=[
            jax.ShapeDtypeStruct((N_TOK, K_CODES), jnp.float32),
            jax.ShapeDtypeStruct((NT_E, 1, K_CODES), jnp.float32),
        ],
        compiler_params=pltpu.CompilerParams(
            dimension_semantics=("parallel",)),
    )(idx2)

    idx = idx2.reshape(N_TOK)
    q = _sc_gather(emb_w, idx)

    qst, loss11, perp11 = pl.pallas_call(
        _final_body,
        grid=(NT_F,),
        in_specs=[
            pl.BlockSpec((TN_F, DIM), lambda i: (i, 0)),
            pl.BlockSpec((TN_F, DIM), lambda i: (i, 0)),
            pl.BlockSpec((NT_E, 1, K_CODES), lambda i: (0, 0, 0)),
        ],
        out_specs=[
            pl.BlockSpec((TN_F, DIM), lambda i: (i, 0)),
            pl.BlockSpec((1, 1), lambda i: (0, 0)),
            pl.BlockSpec((1, 1), lambda i: (0, 0)),
        ],
        out_shape=[
            jax.ShapeDtypeStruct((N_TOK, DIM), jnp.float32),
            jax.ShapeDtypeStruct((1, 1), jnp.float32),
            jax.ShapeDtypeStruct((1, 1), jnp.float32),
        ],
        scratch_shapes=[pltpu.SMEM((1, 1), jnp.float32)],
        compiler_params=pltpu.CompilerParams(
            dimension_semantics=("arbitrary",)),
    )(flat, q, pc)

    loss = loss11[0, 0]
    perplexity = perp11[0, 0]
    quantized_out = jnp.transpose(qst.reshape(in_shape), (0, 3, 1, 2))
    indices = idx.reshape(in_shape[0], in_shape[1], in_shape[2])[:, None, :, :]
    return (loss, quantized_out, perplexity, enc, indices)


# speedup vs baseline: 1.3108x; 1.2907x over previous
import functools

import jax
import jax.numpy as jnp
from jax import lax
from jax.experimental import pallas as pl
from jax.experimental.pallas import tpu as pltpu
from jax.experimental.pallas import tpu_sc as plsc

K_CODES = 8192
DIM = 256
N_TOK = 8192
COMMIT = 0.25

TN = 256
TK = 512
NT_E = N_TOK // TN
TN_F = 512
NT_F = N_TOK // TN_F


def _argmin_body(x_ref, embt_ref, xsq_ref, esq_ref, idx_ref, enc_ref, pc_ref):
    x2 = x_ref[...] + x_ref[...]
    xsq = xsq_ref[...]
    best = jnp.full((TN, 1), jnp.inf, dtype=jnp.float32)
    bidxf = jnp.zeros((TN, 1), dtype=jnp.float32)
    iif = lax.broadcasted_iota(jnp.int32, (TN, TK), 1).astype(jnp.float32)
    for j in range(K_CODES // TK):
        e = embt_ref[j * TK:(j + 1) * TK, :]
        mm2 = lax.dot_general(x2, e, (((1,), (1,)), ((), ())),
                              preferred_element_type=jnp.float32)
        d = (xsq + esq_ref[:, j * TK:(j + 1) * TK]) - mm2
        lmin = jnp.min(d, axis=1, keepdims=True)
        largf = jnp.min(jnp.where(d == lmin, iif, 3.0e7), axis=1,
                        keepdims=True) + float(j * TK)
        upd = lmin < best
        bidxf = jnp.where(upd, largf, bidxf)
        best = jnp.where(upd, lmin, best)
    bidx = bidxf.astype(jnp.int32)
    idx_ref[...] = bidx
    ch = 1024
    for k in range(K_CODES // ch):
        ii = lax.broadcasted_iota(jnp.int32, (TN, ch), 1) + k * ch
        enc = (ii == bidx).astype(jnp.float32)
        enc_ref[:, k * ch:(k + 1) * ch] = enc
        pc_ref[0, 0, k * ch:(k + 1) * ch] = jnp.sum(enc, axis=0)


def _final_body(x_ref, q_ref, pc_ref, qst_ref, loss_ref, perp_ref, acc_ref):
    t = pl.program_id(0)
    x = x_ref[...]
    q = q_ref[...]
    diff = q - x
    qst_ref[...] = x + diff
    psum = jnp.sum(diff * diff)
    prev = jnp.where(t == 0, 0.0, acc_ref[0, 0])
    acc_ref[0, 0] = prev + psum

    @pl.when(t == NT_F - 1)
    def _():
        m = acc_ref[0, 0] * (1.0 / (N_TOK * DIM))
        loss_ref[...] = (m + COMMIT * m)[None, None]
        pc = pc_ref[...].reshape(NT_E, K_CODES)
        counts = jnp.sum(pc, axis=0)
        p = counts * (1.0 / N_TOK)
        ent = jnp.sum(p * jnp.log(p + 1e-10))
        perp_ref[...] = jnp.exp(-ent)[None, None]


_NW = 32
_B_PER_W = N_TOK // _NW


def _sc_gather(emb_w, idx):
    mesh = plsc.VectorSubcoreMesh(core_axis_name="c", subcore_axis_name="s")

    @functools.partial(
        pl.kernel, mesh=mesh,
        out_type=jax.ShapeDtypeStruct((N_TOK, DIM), jnp.float32),
        scratch_types=[
            pltpu.VMEM((_B_PER_W,), jnp.int32),
            pltpu.VMEM((_B_PER_W, DIM), jnp.float32),
            pltpu.SemaphoreType.DMA,
        ],
    )
    def k(table_hbm, idx_hbm, out_hbm, idx_v, rows_v, sem):
        wid = lax.axis_index("s") * 2 + lax.axis_index("c")
        base = wid * _B_PER_W
        pltpu.sync_copy(idx_hbm.at[pl.ds(base, _B_PER_W)], idx_v)
        pltpu.async_copy(table_hbm.at[idx_v], rows_v, sem).wait()
        pltpu.sync_copy(rows_v, out_hbm.at[pl.ds(base, _B_PER_W)])

    return k(emb_w, idx)


def kernel(inputs, emb_w):
    x = jnp.transpose(inputs, (0, 2, 3, 1))
    in_shape = x.shape
    flat = x.reshape(-1, DIM)
    xsq = jnp.sum(flat ** 2, axis=1, keepdims=True)
    esq = jnp.sum(emb_w ** 2, axis=1)[None, :]

    idx2, enc, pc = pl.pallas_call(
        _argmin_body,
        grid=(N_TOK // TN,),
        in_specs=[
            pl.BlockSpec((TN, DIM), lambda i: (i, 0)),
            pl.BlockSpec((K_CODES, DIM), lambda i: (0, 0)),
            pl.BlockSpec((TN, 1), lambda i: (i, 0)),
            pl.BlockSpec((1, K_CODES), lambda i: (0, 0)),
        ],
        out_specs=[
            pl.BlockSpec((TN, 1), lambda i: (i, 0)),
            pl.BlockSpec((TN, K_CODES), lambda i: (i, 0)),
            pl.BlockSpec((1, 1, K_CODES), lambda i: (i, 0, 0)),
        ],
        out_shape=[
            jax.ShapeDtypeStruct((N_TOK, 1), jnp.int32),
            jax.ShapeDtypeStruct((N_TOK, K_CODES), jnp.float32),
            jax.ShapeDtypeStruct((NT_E, 1, K_CODES), jnp.float32),
        ],
        compiler_params=pltpu.CompilerParams(
            dimension_semantics=("parallel",)),
    )(flat, emb_w, xsq, esq)

    idx = idx2.reshape(N_TOK)
    q = _sc_gather(emb_w, idx)

    qst, loss11, perp11 = pl.pallas_call(
        _final_body,
        grid=(NT_F,),
        in_specs=[
            pl.BlockSpec((TN_F, DIM), lambda i: (i, 0)),
            pl.BlockSpec((TN_F, DIM), lambda i: (i, 0)),
            pl.BlockSpec((NT_E, 1, K_CODES), lambda i: (0, 0, 0)),
        ],
        out_specs=[
            pl.BlockSpec((TN_F, DIM), lambda i: (i, 0)),
            pl.BlockSpec((1, 1), lambda i: (0, 0)),
            pl.BlockSpec((1, 1), lambda i: (0, 0)),
        ],
        out_shape=[
            jax.ShapeDtypeStruct((N_TOK, DIM), jnp.float32),
            jax.ShapeDtypeStruct((1, 1), jnp.float32),
            jax.ShapeDtypeStruct((1, 1), jnp.float32),
        ],
        scratch_shapes=[pltpu.SMEM((1, 1), jnp.float32)],
        compiler_params=pltpu.CompilerParams(
            dimension_semantics=("arbitrary",)),
    )(flat, q, pc)

    loss = loss11[0, 0]
    perplexity = perp11[0, 0]
    quantized_out = jnp.transpose(qst.reshape(in_shape), (0, 3, 1, 2))
    indices = idx.reshape(in_shape[0], in_shape[1], in_shape[2])[:, None, :, :]
    return (loss, quantized_out, perplexity, enc, indices)


# speedup vs baseline: 1.3440x; 1.0253x over previous
import functools

import jax
import jax.numpy as jnp
from jax import lax
from jax.experimental import pallas as pl
from jax.experimental.pallas import tpu as pltpu
from jax.experimental.pallas import tpu_sc as plsc

K_CODES = 8192
DIM = 256
N_TOK = 8192
COMMIT = 0.25

TN = 256
TK = 512
NT_E = N_TOK // TN
TN_F = 1024
NT_F = N_TOK // TN_F


def _argmin_body(x_ref, embt_ref, xsq_ref, esq_ref, idx_ref, enc_ref, pc_ref):
    x2 = x_ref[...] + x_ref[...]
    xsq = xsq_ref[...]
    best = jnp.full((TN, 1), jnp.inf, dtype=jnp.float32)
    btf = jnp.zeros((TN, 1), dtype=jnp.float32)
    dwin = jnp.zeros((TN, TK), dtype=jnp.float32)
    for j in range(K_CODES // TK):
        e = embt_ref[j * TK:(j + 1) * TK, :]
        mm2 = lax.dot_general(x2, e, (((1,), (1,)), ((), ())),
                              preferred_element_type=jnp.float32)
        d = (xsq + esq_ref[:, j * TK:(j + 1) * TK]) - mm2
        lmin = jnp.min(d, axis=1, keepdims=True)
        upd = lmin < best
        dwin = jnp.where(upd, d, dwin)
        btf = jnp.where(upd, float(j * TK), btf)
        best = jnp.where(upd, lmin, best)
    iif = lax.broadcasted_iota(jnp.int32, (TN, TK), 1).astype(jnp.float32)
    bidxf = jnp.min(jnp.where(dwin == best, iif, 3.0e7), axis=1,
                    keepdims=True) + btf
    bidx = bidxf.astype(jnp.int32)
    idx_ref[...] = bidx
    ch = 1024
    for k in range(K_CODES // ch):
        ii = lax.broadcasted_iota(jnp.int32, (TN, ch), 1) + k * ch
        enc = (ii == bidx).astype(jnp.float32)
        enc_ref[:, k * ch:(k + 1) * ch] = enc
        pc_ref[0, 0, k * ch:(k + 1) * ch] = jnp.sum(enc, axis=0)


def _final_body(x_ref, q_ref, pc_ref, qst_ref, loss_ref, perp_ref, acc_ref):
    t = pl.program_id(0)
    x = x_ref[...]
    q = q_ref[...]
    diff = q - x
    qst_ref[...] = jnp.transpose(x + diff)[None]
    psum = jnp.sum(diff * diff)
    prev = jnp.where(t == 0, 0.0, acc_ref[0, 0])
    acc_ref[0, 0] = prev + psum

    @pl.when(t == NT_F - 1)
    def _():
        m = acc_ref[0, 0] * (1.0 / (N_TOK * DIM))
        loss_ref[...] = (m + COMMIT * m)[None, None]
        pc = pc_ref[...].reshape(NT_E, K_CODES)
        counts = jnp.sum(pc, axis=0)
        p = counts * (1.0 / N_TOK)
        ent = jnp.sum(p * jnp.log(p + 1e-10))
        perp_ref[...] = jnp.exp(-ent)[None, None]


_NW = 32
_B_PER_W = N_TOK // _NW


def _sc_gather(emb_w, idx):
    mesh = plsc.VectorSubcoreMesh(core_axis_name="c", subcore_axis_name="s")

    @functools.partial(
        pl.kernel, mesh=mesh,
        out_type=jax.ShapeDtypeStruct((N_TOK, DIM), jnp.float32),
        scratch_types=[
            pltpu.VMEM((_B_PER_W,), jnp.int32),
            pltpu.VMEM((_B_PER_W, DIM), jnp.float32),
            pltpu.SemaphoreType.DMA,
        ],
    )
    def k(table_hbm, idx_hbm, out_hbm, idx_v, rows_v, sem):
        wid = lax.axis_index("s") * 2 + lax.axis_index("c")
        base = wid * _B_PER_W
        pltpu.sync_copy(idx_hbm.at[pl.ds(base, _B_PER_W)], idx_v)
        pltpu.async_copy(table_hbm.at[idx_v], rows_v, sem).wait()
        pltpu.sync_copy(rows_v, out_hbm.at[pl.ds(base, _B_PER_W)])

    return k(emb_w, idx)


def kernel(inputs, emb_w):
    x = jnp.transpose(inputs, (0, 2, 3, 1))
    in_shape = x.shape
    flat = x.reshape(-1, DIM)
    xsq = jnp.sum(flat ** 2, axis=1, keepdims=True)
    esq = jnp.sum(emb_w ** 2, axis=1)[None, :]

    idx2, enc, pc = pl.pallas_call(
        _argmin_body,
        grid=(N_TOK // TN,),
        in_specs=[
            pl.BlockSpec((TN, DIM), lambda i: (i, 0)),
            pl.BlockSpec((K_CODES, DIM), lambda i: (0, 0)),
            pl.BlockSpec((TN, 1), lambda i: (i, 0)),
            pl.BlockSpec((1, K_CODES), lambda i: (0, 0)),
        ],
        out_specs=[
            pl.BlockSpec((TN, 1), lambda i: (i, 0)),
            pl.BlockSpec((TN, K_CODES), lambda i: (i, 0)),
            pl.BlockSpec((1, 1, K_CODES), lambda i: (i, 0, 0)),
        ],
        out_shape=[
            jax.ShapeDtypeStruct((N_TOK, 1), jnp.int32),
            jax.ShapeDtypeStruct((N_TOK, K_CODES), jnp.float32),
            jax.ShapeDtypeStruct((NT_E, 1, K_CODES), jnp.float32),
        ],
        compiler_params=pltpu.CompilerParams(
            dimension_semantics=("parallel",)),
    )(flat, emb_w, xsq, esq)

    idx = idx2.reshape(N_TOK)
    q = _sc_gather(emb_w, idx)

    qst, loss11, perp11 = pl.pallas_call(
        _final_body,
        grid=(NT_F,),
        in_specs=[
            pl.BlockSpec((TN_F, DIM), lambda i: (i, 0)),
            pl.BlockSpec((TN_F, DIM), lambda i: (i, 0)),
            pl.BlockSpec((NT_E, 1, K_CODES), lambda i: (0, 0, 0)),
        ],
        out_specs=[
            pl.BlockSpec((1, DIM, TN_F), lambda i: (i, 0, 0)),
            pl.BlockSpec((1, 1), lambda i: (0, 0)),
            pl.BlockSpec((1, 1), lambda i: (0, 0)),
        ],
        out_shape=[
            jax.ShapeDtypeStruct((NT_F, DIM, TN_F), jnp.float32),
            jax.ShapeDtypeStruct((1, 1), jnp.float32),
            jax.ShapeDtypeStruct((1, 1), jnp.float32),
        ],
        scratch_shapes=[pltpu.SMEM((1, 1), jnp.float32)],
        compiler_params=pltpu.CompilerParams(
            dimension_semantics=("arbitrary",)),
    )(flat, q, pc)

    loss = loss11[0, 0]
    perplexity = perp11[0, 0]
    quantized_out = qst.reshape(in_shape[0], DIM, in_shape[1], in_shape[2])
    indices = idx.reshape(in_shape[0], in_shape[1], in_shape[2])[:, None, :, :]
    return (loss, quantized_out, perplexity, enc, indices)


# speedup vs baseline: 1.4671x; 1.0916x over previous
import functools

import jax
import jax.numpy as jnp
from jax import lax
from jax.experimental import pallas as pl
from jax.experimental.pallas import tpu as pltpu
from jax.experimental.pallas import tpu_sc as plsc

K_CODES = 8192
DIM = 256
N_TOK = 8192
COMMIT = 0.25

TN = 256
TK = 512
NT_E = N_TOK // TN
TN_F = 1024
NT_F = N_TOK // TN_F


def _argmin_body(x_ref, embt_ref, xsq_ref, esq_ref, idx_ref, enc_ref, pc_ref):
    x2 = x_ref[...] + x_ref[...]
    xsq = xsq_ref[...]
    best = jnp.full((TN, 1), jnp.inf, dtype=jnp.float32)
    btf = jnp.zeros((TN, 1), dtype=jnp.float32)
    dwin = jnp.zeros((TN, TK), dtype=jnp.float32)
    for j in range(K_CODES // TK):
        e = embt_ref[j * TK:(j + 1) * TK, :]
        mm2 = lax.dot_general(x2, e, (((1,), (1,)), ((), ())),
                              preferred_element_type=jnp.float32)
        d = (xsq + esq_ref[:, j * TK:(j + 1) * TK]) - mm2
        lmin = jnp.min(d, axis=1, keepdims=True)
        upd = lmin < best
        dwin = jnp.where(upd, d, dwin)
        btf = jnp.where(upd, float(j * TK), btf)
        best = jnp.where(upd, lmin, best)
    iif = lax.broadcasted_iota(jnp.int32, (TN, TK), 1).astype(jnp.float32)
    bidxf = jnp.min(jnp.where(dwin == best, iif, 3.0e7), axis=1,
                    keepdims=True) + btf
    bidx = bidxf.astype(jnp.int32)
    idx_ref[...] = bidx
    ones = jnp.ones((8, TN), dtype=jnp.float32)
    ch = 1024
    for k in range(K_CODES // ch):
        ii = lax.broadcasted_iota(jnp.int32, (TN, ch), 1) + k * ch
        enc = (ii == bidx).astype(jnp.float32)
        enc_ref[:, k * ch:(k + 1) * ch] = enc
        pc_ref[0, 0, k * ch:(k + 1) * ch] = jnp.dot(
            ones, enc, preferred_element_type=jnp.float32)[0]


def _final_body(x_ref, q_ref, pc_ref, qst_ref, loss_ref, perp_ref, acc_ref):
    t = pl.program_id(0)
    x = x_ref[...]
    q = q_ref[...]
    diff = q - x
    qst_ref[...] = jnp.transpose(x + diff)[None]
    psum = jnp.sum(diff * diff)
    prev = jnp.where(t == 0, 0.0, acc_ref[0, 0])
    acc_ref[0, 0] = prev + psum

    @pl.when(t == NT_F - 1)
    def _():
        m = acc_ref[0, 0] * (1.0 / (N_TOK * DIM))
        loss_ref[...] = (m + COMMIT * m)[None, None]
        pc = pc_ref[...].reshape(NT_E, K_CODES)
        counts = jnp.sum(pc, axis=0)
        p = counts * (1.0 / N_TOK)
        ent = jnp.sum(p * jnp.log(p + 1e-10))
        perp_ref[...] = jnp.exp(-ent)[None, None]


_NW = 32
_B_PER_W = N_TOK // _NW


def _sc_gather(emb_w, idx):
    mesh = plsc.VectorSubcoreMesh(core_axis_name="c", subcore_axis_name="s")

    @functools.partial(
        pl.kernel, mesh=mesh,
        out_type=jax.ShapeDtypeStruct((N_TOK, DIM), jnp.float32),
        scratch_types=[
            pltpu.VMEM((_B_PER_W,), jnp.int32),
            pltpu.VMEM((_B_PER_W, DIM), jnp.float32),
            pltpu.SemaphoreType.DMA,
        ],
    )
    def k(table_hbm, idx_hbm, out_hbm, idx_v, rows_v, sem):
        wid = lax.axis_index("s") * 2 + lax.axis_index("c")
        base = wid * _B_PER_W
        pltpu.sync_copy(idx_hbm.at[pl.ds(base, _B_PER_W)], idx_v)
        pltpu.async_copy(table_hbm.at[idx_v], rows_v, sem).wait()
        pltpu.sync_copy(rows_v, out_hbm.at[pl.ds(base, _B_PER_W)])

    return k(emb_w, idx)


def kernel(inputs, emb_w):
    x = jnp.transpose(inputs, (0, 2, 3, 1))
    in_shape = x.shape
    flat = x.reshape(-1, DIM)
    xsq = jnp.sum(flat ** 2, axis=1, keepdims=True)
    esq = jnp.sum(emb_w ** 2, axis=1)[None, :]

    idx2, enc, pc = pl.pallas_call(
        _argmin_body,
        grid=(N_TOK // TN,),
        in_specs=[
            pl.BlockSpec((TN, DIM), lambda i: (i, 0)),
            pl.BlockSpec((K_CODES, DIM), lambda i: (0, 0)),
            pl.BlockSpec((TN, 1), lambda i: (i, 0)),
            pl.BlockSpec((1, K_CODES), lambda i: (0, 0)),
        ],
        out_specs=[
            pl.BlockSpec((TN, 1), lambda i: (i, 0)),
            pl.BlockSpec((TN, K_CODES), lambda i: (i, 0)),
            pl.BlockSpec((1, 1, K_CODES), lambda i: (i, 0, 0)),
        ],
        out_shape=[
            jax.ShapeDtypeStruct((N_TOK, 1), jnp.int32),
            jax.ShapeDtypeStruct((N_TOK, K_CODES), jnp.float32),
            jax.ShapeDtypeStruct((NT_E, 1, K_CODES), jnp.float32),
        ],
        compiler_params=pltpu.CompilerParams(
            dimension_semantics=("parallel",)),
    )(flat, emb_w, xsq, esq)

    idx = idx2.reshape(N_TOK)
    q = _sc_gather(emb_w, idx)

    qst, loss11, perp11 = pl.pallas_call(
        _final_body,
        grid=(NT_F,),
        in_specs=[
            pl.BlockSpec((TN_F, DIM), lambda i: (i, 0)),
            pl.BlockSpec((TN_F, DIM), lambda i: (i, 0)),
            pl.BlockSpec((NT_E, 1, K_CODES), lambda i: (0, 0, 0)),
        ],
        out_specs=[
            pl.BlockSpec((1, DIM, TN_F), lambda i: (i, 0, 0)),
            pl.BlockSpec((1, 1), lambda i: (0, 0)),
            pl.BlockSpec((1, 1), lambda i: (0, 0)),
        ],
        out_shape=[
            jax.ShapeDtypeStruct((NT_F, DIM, TN_F), jnp.float32),
            jax.ShapeDtypeStruct((1, 1), jnp.float32),
            jax.ShapeDtypeStruct((1, 1), jnp.float32),
        ],
        scratch_shapes=[pltpu.SMEM((1, 1), jnp.float32)],
        compiler_params=pltpu.CompilerParams(
            dimension_semantics=("arbitrary",)),
    )(flat, q, pc)

    loss = loss11[0, 0]
    perplexity = perp11[0, 0]
    quantized_out = qst.reshape(in_shape[0], DIM, in_shape[1], in_shape[2])
    indices = idx.reshape(in_shape[0], in_shape[1], in_shape[2])[:, None, :, :]
    return (loss, quantized_out, perplexity, enc, indices)


# speedup vs baseline: 1.4749x; 1.0053x over previous
import functools

import jax
import jax.numpy as jnp
from jax import lax
from jax.experimental import pallas as pl
from jax.experimental.pallas import tpu as pltpu
from jax.experimental.pallas import tpu_sc as plsc

K_CODES = 8192
DIM = 256
N_TOK = 8192
COMMIT = 0.25

TN = 256
TK = 512
NT_E = N_TOK // TN
TN_F = 1024
NT_F = N_TOK // TN_F


def _argmin_body(x_ref, embt_ref, esq_ref, idx_ref, enc_ref, pc_ref):
    x = jnp.transpose(x_ref[0])
    x2 = x + x
    xsq = jnp.sum(x * x, axis=1, keepdims=True)
    best = jnp.full((TN, 1), jnp.inf, dtype=jnp.float32)
    btf = jnp.zeros((TN, 1), dtype=jnp.float32)
    dwin = jnp.zeros((TN, TK), dtype=jnp.float32)
    for j in range(K_CODES // TK):
        e = embt_ref[j * TK:(j + 1) * TK, :]
        mm2 = lax.dot_general(x2, e, (((1,), (1,)), ((), ())),
                              preferred_element_type=jnp.float32)
        d = (xsq + esq_ref[:, j * TK:(j + 1) * TK]) - mm2
        lmin = jnp.min(d, axis=1, keepdims=True)
        upd = lmin < best
        dwin = jnp.where(upd, d, dwin)
        btf = jnp.where(upd, float(j * TK), btf)
        best = jnp.where(upd, lmin, best)
    iif = lax.broadcasted_iota(jnp.int32, (TN, TK), 1).astype(jnp.float32)
    bidxf = jnp.min(jnp.where(dwin == best, iif, 3.0e7), axis=1,
                    keepdims=True) + btf
    bidx = bidxf.astype(jnp.int32)
    idx_ref[...] = bidx
    ones = jnp.ones((8, TN), dtype=jnp.float32)
    ch = 1024
    for k in range(K_CODES // ch):
        ii = lax.broadcasted_iota(jnp.int32, (TN, ch), 1) + k * ch
        enc = (ii == bidx).astype(jnp.float32)
        enc_ref[:, k * ch:(k + 1) * ch] = enc
        pc_ref[0, 0, k * ch:(k + 1) * ch] = jnp.dot(
            ones, enc, preferred_element_type=jnp.float32)[0]


def _final_body(x_ref, q_ref, pc_ref, qst_ref, loss_ref, perp_ref, acc_ref):
    t = pl.program_id(0)
    xt = x_ref[0]
    qt = jnp.transpose(q_ref[...])
    diff = qt - xt
    qst_ref[...] = (xt + diff)[None]
    psum = jnp.sum(diff * diff)
    prev = jnp.where(t == 0, 0.0, acc_ref[0, 0])
    acc_ref[0, 0] = prev + psum

    @pl.when(t == NT_F - 1)
    def _():
        m = acc_ref[0, 0] * (1.0 / (N_TOK * DIM))
        loss_ref[...] = (m + COMMIT * m)[None, None]
        pc = pc_ref[...].reshape(NT_E, K_CODES)
        counts = jnp.sum(pc, axis=0)
        p = counts * (1.0 / N_TOK)
        ent = jnp.sum(p * jnp.log(p + 1e-10))
        perp_ref[...] = jnp.exp(-ent)[None, None]


_NW = 32
_B_PER_W = N_TOK // _NW


def _sc_gather(emb_w, idx):
    mesh = plsc.VectorSubcoreMesh(core_axis_name="c", subcore_axis_name="s")

    @functools.partial(
        pl.kernel, mesh=mesh,
        out_type=jax.ShapeDtypeStruct((N_TOK, DIM), jnp.float32),
        scratch_types=[
            pltpu.VMEM((_B_PER_W,), jnp.int32),
            pltpu.VMEM((_B_PER_W, DIM), jnp.float32),
            pltpu.SemaphoreType.DMA,
        ],
    )
    def k(table_hbm, idx_hbm, out_hbm, idx_v, rows_v, sem):
        wid = lax.axis_index("s") * 2 + lax.axis_index("c")
        base = wid * _B_PER_W
        pltpu.sync_copy(idx_hbm.at[pl.ds(base, _B_PER_W)], idx_v)
        pltpu.async_copy(table_hbm.at[idx_v], rows_v, sem).wait()
        pltpu.sync_copy(rows_v, out_hbm.at[pl.ds(base, _B_PER_W)])

    return k(emb_w, idx)


def kernel(inputs, emb_w):
    nb, _, nh, nw = inputs.shape
    in3 = inputs.reshape(nb, DIM, nh * nw)
    esq = jnp.sum(emb_w ** 2, axis=1)[None, :]
    tpb = (nh * nw) // TN

    idx2, enc, pc = pl.pallas_call(
        _argmin_body,
        grid=(N_TOK // TN,),
        in_specs=[
            pl.BlockSpec((1, DIM, TN), lambda i: (i // tpb, 0, i % tpb)),
            pl.BlockSpec((K_CODES, DIM), lambda i: (0, 0)),
            pl.BlockSpec((1, K_CODES), lambda i: (0, 0)),
        ],
        out_specs=[
            pl.BlockSpec((TN, 1), lambda i: (i, 0)),
            pl.BlockSpec((TN, K_CODES), lambda i: (i, 0)),
            pl.BlockSpec((1, 1, K_CODES), lambda i: (i, 0, 0)),
        ],
        out_shape=[
            jax.ShapeDtypeStruct((N_TOK, 1), jnp.int32),
            jax.ShapeDtypeStruct((N_TOK, K_CODES), jnp.float32),
            jax.ShapeDtypeStruct((NT_E, 1, K_CODES), jnp.float32),
        ],
        compiler_params=pltpu.CompilerParams(
            dimension_semantics=("parallel",)),
    )(in3, emb_w, esq)

    idx = idx2.reshape(N_TOK)
    q = _sc_gather(emb_w, idx)

    qst, loss11, perp11 = pl.pallas_call(
        _final_body,
        grid=(NT_F,),
        in_specs=[
            pl.BlockSpec((1, DIM, TN_F), lambda i: (i, 0, 0)),
            pl.BlockSpec((TN_F, DIM), lambda i: (i, 0)),
            pl.BlockSpec((NT_E, 1, K_CODES), lambda i: (0, 0, 0)),
        ],
        out_specs=[
            pl.BlockSpec((1, DIM, TN_F), lambda i: (i, 0, 0)),
            pl.BlockSpec((1, 1), lambda i: (0, 0)),
            pl.BlockSpec((1, 1), lambda i: (0, 0)),
        ],
        out_shape=[
            jax.ShapeDtypeStruct((NT_F, DIM, TN_F), jnp.float32),
            jax.ShapeDtypeStruct((1, 1), jnp.float32),
            jax.ShapeDtypeStruct((1, 1), jnp.float32),
        ],
        scratch_shapes=[pltpu.SMEM((1, 1), jnp.float32)],
        compiler_params=pltpu.CompilerParams(
            dimension_semantics=("arbitrary",)),
    )(in3, q, pc)

    loss = loss11[0, 0]
    perplexity = perp11[0, 0]
    quantized_out = qst.reshape(nb, DIM, nh, nw)
    indices = idx.reshape(nb, nh, nw)[:, None, :, :]
    return (loss, quantized_out, perplexity, enc, indices)
